# Initial kernel scaffold; baseline (speedup 1.0000x reference)
import jax
import jax.numpy as jnp
from jax.experimental import pallas as pl


def kernel(x, norm1_w, norm2_w, Wq, Wk, Wv, Wo, Wg, Wmoe, W1, W2):
    raise NotImplementedError("write your pallas kernel here")


# speedup vs baseline: 1.1021x; 1.1021x over previous
import functools

import jax
import jax.numpy as jnp
from jax import lax
from jax.experimental import pallas as pl
from jax.experimental.pallas import tpu as pltpu
from jax.experimental.pallas import tpu_sc as plsc

_D, _H, _E, _KK, _FF = 768, 12, 8, 2, 2048
_B, _N = 1, 2048
_HD = _D // _H
_TB = 256
_QB = 512
_BLK = 128
_P = _N * _KK + _E * _BLK
_NB = _P // _BLK
_NEG = -1e30
_BF = jnp.bfloat16


def _dot_t(a, b):
    return lax.dot_general(a.astype(_BF), b.astype(_BF),
                           (((1,), (1,)), ((), ())),
                           preferred_element_type=jnp.float32)


def _rms(x, w):
    return x / jnp.sqrt(jnp.mean(x * x, axis=1, keepdims=True) + 1e-6) * w


def _pre_body(x_ref, n1_ref, wq_ref, wk_ref, wv_ref, wg_ref,
              q_ref, k_ref, v_ref, g_ref):
    h = _rms(x_ref[...], n1_ref[...])
    q_ref[...] = _dot_t(h, wq_ref[...]).astype(_BF)
    k_ref[...] = _dot_t(h, wk_ref[...]).astype(_BF)
    v_ref[...] = _dot_t(h, wv_ref[...]).astype(_BF)
    g_ref[...] = jax.nn.sigmoid(_dot_t(h, wg_ref[...]))


def _attn_body(q_ref, k_ref, v_ref, o_ref):
    s = _dot_t(q_ref[0], k_ref[0]) * 0.125
    m = jnp.max(s, axis=1, keepdims=True)
    p = jnp.exp(s - m)
    l = jnp.sum(p, axis=1, keepdims=True)
    o = lax.dot_general(p.astype(_BF), v_ref[0], (((1,), (0,)), ((), ())),
                        preferred_element_type=jnp.float32)
    o_ref[0] = (o / l).astype(_BF)


def _post_body(a_ref, g_ref, x_ref, n2_ref, wo_ref, wm_ref,
               x1_ref, h2_ref, lg_ref):
    out = _dot_t(a_ref[...], wo_ref[...])
    x1 = x_ref[...] + out * g_ref[...]
    x1_ref[...] = x1
    h2 = _rms(x1, n2_ref[...])
    h2_ref[...] = h2
    lg_ref[...] = _dot_t(h2, wm_ref[...])


def _router_body(lg_ref, pos_ref, w_ref, meta_ref):
    lg = lg_ref[...]
    ji = lax.broadcasted_iota(jnp.int32, (_N, 128), 1)
    l1 = jnp.where(ji < _E, lg, _NEG)
    m1 = jnp.max(l1, axis=1, keepdims=True)
    i1 = jnp.min(jnp.where(l1 == m1, ji, 127), axis=1, keepdims=True)
    l2 = jnp.where(ji == i1, _NEG, l1)
    m2 = jnp.max(l2, axis=1, keepdims=True)
    i2 = jnp.min(jnp.where(l2 == m2, ji, 127), axis=1, keepdims=True)
    e2 = jnp.exp(m2 - m1)
    w1 = 1.0 / (1.0 + e2)
    w2 = e2 / (1.0 + e2)
    a0 = (ji == i1).astype(jnp.int32)
    a1 = (ji == i2).astype(jnp.int32)

    def shift_rows(a, sh):
        return jnp.concatenate(
            [jnp.zeros((sh, 128), a.dtype), a[: _N - sh]], axis=0)

    def cumsum_ex_rows(a):
        a = shift_rows(a, 1)
        sh = 1
        while sh < _N:
            a = a + shift_rows(a, sh)
            sh *= 2
        return a

    c0x = cumsum_ex_rows(a0)
    c1x = cumsum_ex_rows(a1)
    c0 = jnp.sum(a0, axis=0, keepdims=True)
    c1 = jnp.sum(a1, axis=0, keepdims=True)
    pc = ((c0 + c1 + (_BLK - 1)) // _BLK) * _BLK

    def shift_lanes(a, sh):
        return jnp.concatenate(
            [jnp.zeros((a.shape[0], sh), a.dtype), a[:, : 128 - sh]], axis=1)

    off = shift_lanes(pc, 1)
    for sh in (1, 2, 4):
        off = off + shift_lanes(off, sh)
    cum_end = off + pc
    pos0 = jnp.sum(a0 * (c0x + off), axis=1, keepdims=True)
    pos1 = jnp.sum(a1 * (c1x + c0 + off), axis=1, keepdims=True)
    pos_ref[...] = jnp.where(ji == 0, pos0, jnp.where(ji == 1, pos1, 0))
    w_ref[...] = jnp.where(ji == 0, w1, jnp.where(ji == 1, w2, 0.0))
    bi = lax.broadcasted_iota(jnp.int32, (128, 128), 0) * _BLK
    ji2 = lax.broadcasted_iota(jnp.int32, (128, 128), 1)
    ge = jnp.where((ji2 < _E) & (bi >= cum_end), 1, 0)
    be = jnp.minimum(jnp.sum(ge, axis=1, keepdims=True), _E - 1)
    meta_ref[...] = jnp.where(ji2 == 0, be, 0)


def _ffn_body(be_ref, x_ref, w1_ref, w2_ref, o_ref):
    h = _dot_t(x_ref[...], w1_ref[0])
    h = h * jax.nn.sigmoid(h)
    o_ref[...] = _dot_t(h, w2_ref[0])


def _comb_body(x1_ref, g0_ref, g1_ref, w_ref, y_ref):
    y_ref[...] = (x1_ref[...]
                  + w_ref[:, 0:1] * g0_ref[...]
                  + w_ref[:, 1:2] * g1_ref[...])


def _sc_scatter(h2, idx):
    @functools.partial(
        pl.kernel,
        mesh=plsc.VectorSubcoreMesh(core_axis_name="c", subcore_axis_name="s"),
        out_type=jax.ShapeDtypeStruct((_P, _D), jnp.float32),
        scratch_types=[
            pltpu.VMEM((128,), jnp.int32),
            pltpu.VMEM((128, _D), jnp.float32),
            pltpu.SemaphoreType.DMA,
        ],
    )
    def k(h2_hbm, idx_hbm, out_hbm, idx_v, rows_v, sem):
        wid = lax.axis_index("s") * 2 + lax.axis_index("c")
        src = (wid % 16) * 128
        pltpu.sync_copy(idx_hbm.at[wid], idx_v)
        pltpu.sync_copy(h2_hbm.at[pl.ds(src, 128)], rows_v)
        pltpu.async_copy(rows_v, out_hbm.at[idx_v], sem).wait()

    return k(h2, idx)


def _sc_gather(f, idx):
    @functools.partial(
        pl.kernel,
        mesh=plsc.VectorSubcoreMesh(core_axis_name="c", subcore_axis_name="s"),
        out_type=jax.ShapeDtypeStruct((_N * _KK, _D), jnp.float32),
        scratch_types=[
            pltpu.VMEM((128,), jnp.int32),
            pltpu.VMEM((128, _D), jnp.float32),
            pltpu.SemaphoreType.DMA,
        ],
    )
    def k(f_hbm, idx_hbm, out_hbm, idx_v, rows_v, sem):
        wid = lax.axis_index("s") * 2 + lax.axis_index("c")
        pltpu.sync_copy(idx_hbm.at[wid], idx_v)
        pltpu.async_copy(f_hbm.at[idx_v], rows_v, sem).wait()
        pltpu.sync_copy(rows_v, out_hbm.at[pl.ds(wid * 128, 128)])

    return k(f, idx)


def kernel(x, norm1_w, norm2_w, Wq, Wk, Wv, Wo, Wg, Wmoe, W1, W2):
    x2d = x.reshape(_N, _D)
    n1 = norm1_w.reshape(1, _D)
    n2 = norm2_w.reshape(1, _D)
    wm_pad = jnp.pad(Wmoe, ((0, 128 - _E), (0, 0)))

    q, k, v, g = pl.pallas_call(
        _pre_body,
        grid=(_N // _TB,),
        in_specs=[
            pl.BlockSpec((_TB, _D), lambda t: (t, 0)),
            pl.BlockSpec((1, _D), lambda t: (0, 0)),
            pl.BlockSpec((_D, _D), lambda t: (0, 0)),
            pl.BlockSpec((_D, _D), lambda t: (0, 0)),
            pl.BlockSpec((_D, _D), lambda t: (0, 0)),
            pl.BlockSpec((_D, _D), lambda t: (0, 0)),
        ],
        out_specs=[pl.BlockSpec((_TB, _D), lambda t: (t, 0))] * 4,
        out_shape=[jax.ShapeDtypeStruct((_N, _D), _BF)] * 3
        + [jax.ShapeDtypeStruct((_N, _D), jnp.float32)],
    )(x2d, n1, Wq, Wk, Wv, Wg)

    def _heads(a):
        return a.reshape(_N, _H, _HD).transpose(1, 0, 2)

    attn3 = pl.pallas_call(
        _attn_body,
        grid=(_H, _N // _QB),
        in_specs=[
            pl.BlockSpec((1, _QB, _HD), lambda h, t: (h, t, 0)),
            pl.BlockSpec((1, _N, _HD), lambda h, t: (h, 0, 0)),
            pl.BlockSpec((1, _N, _HD), lambda h, t: (h, 0, 0)),
        ],
        out_specs=pl.BlockSpec((1, _QB, _HD), lambda h, t: (h, t, 0)),
        out_shape=jax.ShapeDtypeStruct((_H, _N, _HD), _BF),
    )(_heads(q), _heads(k), _heads(v))
    attn = attn3.transpose(1, 0, 2).reshape(_N, _D)

    x1, h2, lg = pl.pallas_call(
        _post_body,
        grid=(_N // _TB,),
        in_specs=[
            pl.BlockSpec((_TB, _D), lambda t: (t, 0)),
            pl.BlockSpec((_TB, _D), lambda t: (t, 0)),
            pl.BlockSpec((_TB, _D), lambda t: (t, 0)),
            pl.BlockSpec((1, _D), lambda t: (0, 0)),
            pl.BlockSpec((_D, _D), lambda t: (0, 0)),
            pl.BlockSpec((128, _D), lambda t: (0, 0)),
        ],
        out_specs=[
            pl.BlockSpec((_TB, _D), lambda t: (t, 0)),
            pl.BlockSpec((_TB, _D), lambda t: (t, 0)),
            pl.BlockSpec((_TB, 128), lambda t: (t, 0)),
        ],
        out_shape=[
            jax.ShapeDtypeStruct((_N, _D), jnp.float32),
            jax.ShapeDtypeStruct((_N, _D), jnp.float32),
            jax.ShapeDtypeStruct((_N, 128), jnp.float32),
        ],
    )(attn, g, x2d, n2, Wo, wm_pad)

    pos_out, w_out, meta = pl.pallas_call(
        _router_body,
        out_shape=[
            jax.ShapeDtypeStruct((_N, 128), jnp.int32),
            jax.ShapeDtypeStruct((_N, 128), jnp.float32),
            jax.ShapeDtypeStruct((128, 128), jnp.int32),
        ],
    )(lg)

    idx = jnp.concatenate([pos_out[:, 0], pos_out[:, 1]]).reshape(32, 128)
    be = meta[: _NB, 0]

    xs = _sc_scatter(h2, idx)

    f = pl.pallas_call(
        _ffn_body,
        grid_spec=pltpu.PrefetchScalarGridSpec(
            num_scalar_prefetch=1,
            grid=(_NB,),
            in_specs=[
                pl.BlockSpec((_BLK, _D), lambda b, be_r: (b, 0)),
                pl.BlockSpec((1, _FF, _D), lambda b, be_r: (be_r[b], 0, 0)),
                pl.BlockSpec((1, _D, _FF), lambda b, be_r: (be_r[b], 0, 0)),
            ],
            out_specs=pl.BlockSpec((_BLK, _D), lambda b, be_r: (b, 0)),
        ),
        out_shape=jax.ShapeDtypeStruct((_P, _D), jnp.float32),
    )(be, xs, W1, W2)

    gth = _sc_gather(f, idx)

    y = pl.pallas_call(
        _comb_body,
        grid=(_N // _TB,),
        in_specs=[
            pl.BlockSpec((_TB, _D), lambda t: (t, 0)),
            pl.BlockSpec((_TB, _D), lambda t: (t, 0)),
            pl.BlockSpec((_TB, _D), lambda t: (t, 0)),
            pl.BlockSpec((_TB, 128), lambda t: (t, 0)),
        ],
        out_specs=pl.BlockSpec((_TB, _D), lambda t: (t, 0)),
        out_shape=jax.ShapeDtypeStruct((_N, _D), jnp.float32),
    )(x1, gth[: _N], gth[_N:], w_out)

    return y.reshape(_B, _N, _D)


# speedup vs baseline: 1.1831x; 1.0735x over previous
import functools

import jax
import jax.numpy as jnp
from jax import lax
from jax.experimental import pallas as pl
from jax.experimental.pallas import tpu as pltpu
from jax.experimental.pallas import tpu_sc as plsc

_D, _H, _E, _KK, _FF = 768, 12, 8, 2, 2048
_B, _N = 1, 2048
_HD = _D // _H
_TB = 256
_QB = 1024
_BLK = 128
_P = _N * _KK + _E * _BLK
_NB = _P // _BLK
_NEG = -1e30
_BF = jnp.bfloat16


def _dot_t(a, b):
    return lax.dot_general(a.astype(_BF), b.astype(_BF),
                           (((1,), (1,)), ((), ())),
                           preferred_element_type=jnp.float32)


def _rms(x, w):
    return x / jnp.sqrt(jnp.mean(x * x, axis=1, keepdims=True) + 1e-6) * w


def _pre_body(x_ref, n1_ref, wq_ref, wk_ref, wv_ref, wg_ref,
              q_ref, k_ref, v_ref, g_ref):
    h = _rms(x_ref[...], n1_ref[...])

    def hm(z):
        return z.reshape(_TB, _H, _HD).transpose(1, 0, 2).astype(_BF)

    q_ref[...] = hm(_dot_t(h, wq_ref[...]))
    k_ref[...] = hm(_dot_t(h, wk_ref[...]))
    v_ref[...] = hm(_dot_t(h, wv_ref[...]))
    g_ref[...] = jax.nn.sigmoid(_dot_t(h, wg_ref[...]))


def _attn_body(q_ref, k_ref, v_ref, o_ref):
    s = _dot_t(q_ref[0], k_ref[0]) * 0.125
    m = jnp.max(s, axis=1, keepdims=True)
    p = jnp.exp(s - m)
    l = jnp.sum(p, axis=1, keepdims=True)
    o = lax.dot_general(p.astype(_BF), v_ref[0], (((1,), (0,)), ((), ())),
                        preferred_element_type=jnp.float32)
    o_ref[0] = (o / l).astype(_BF)


def _post_body(a_ref, g_ref, x_ref, n2_ref, wo_ref, wm_ref,
               x1_ref, h2_ref, pos_ref, w_ref, meta_ref):
    a3 = a_ref[...]
    a = a3.transpose(1, 0, 2).reshape(_N, _D)
    out = _dot_t(a, wo_ref[...])
    x1 = x_ref[...] + out * g_ref[...]
    x1_ref[...] = x1
    h2 = _rms(x1, n2_ref[...])
    h2_ref[...] = h2
    lg = _dot_t(h2, wm_ref[...])
    ji = lax.broadcasted_iota(jnp.int32, (_N, 128), 1)
    l1 = jnp.where(ji < _E, lg, _NEG)
    m1 = jnp.max(l1, axis=1, keepdims=True)
    i1 = jnp.min(jnp.where(l1 == m1, ji, 127), axis=1, keepdims=True)
    l2 = jnp.where(ji == i1, _NEG, l1)
    m2 = jnp.max(l2, axis=1, keepdims=True)
    i2 = jnp.min(jnp.where(l2 == m2, ji, 127), axis=1, keepdims=True)
    e2 = jnp.exp(m2 - m1)
    w1 = 1.0 / (1.0 + e2)
    w2 = e2 / (1.0 + e2)
    a0 = (ji == i1).astype(jnp.int32)
    a1 = (ji == i2).astype(jnp.int32)

    def shift_rows(a, sh):
        return jnp.concatenate(
            [jnp.zeros((sh, 128), a.dtype), a[: _N - sh]], axis=0)

    def cumsum_ex_rows(a):
        a = shift_rows(a, 1)
        sh = 1
        while sh < _N:
            a = a + shift_rows(a, sh)
            sh *= 2
        return a

    c0x = cumsum_ex_rows(a0)
    c1x = cumsum_ex_rows(a1)
    c0 = jnp.sum(a0, axis=0, keepdims=True)
    c1 = jnp.sum(a1, axis=0, keepdims=True)
    pc = ((c0 + c1 + (_BLK - 1)) // _BLK) * _BLK

    def shift_lanes(a, sh):
        return jnp.concatenate(
            [jnp.zeros((a.shape[0], sh), a.dtype), a[:, : 128 - sh]], axis=1)

    off = shift_lanes(pc, 1)
    for sh in (1, 2, 4):
        off = off + shift_lanes(off, sh)
    cum_end = off + pc
    pos0 = jnp.sum(a0 * (c0x + off), axis=1, keepdims=True)
    pos1 = jnp.sum(a1 * (c1x + c0 + off), axis=1, keepdims=True)
    pos_ref[...] = jnp.where(ji == 0, pos0, jnp.where(ji == 1, pos1, 0))
    w_ref[...] = jnp.where(ji == 0, w1, jnp.where(ji == 1, w2, 0.0))
    bi = lax.broadcasted_iota(jnp.int32, (128, 128), 0) * _BLK
    ji2 = lax.broadcasted_iota(jnp.int32, (128, 128), 1)
    ge = jnp.where((ji2 < _E) & (bi >= cum_end), 1, 0)
    be = jnp.minimum(jnp.sum(ge, axis=1, keepdims=True), _E - 1)
    meta_ref[...] = jnp.where(ji2 == 0, be, 0)


def _ffn_body(be_ref, x_ref, w1_ref, w2_ref, o_ref):
    h = _dot_t(x_ref[...], w1_ref[0])
    h = h * jax.nn.sigmoid(h)
    o_ref[...] = _dot_t(h, w2_ref[0])


def _comb_body(x1_ref, g0_ref, g1_ref, w_ref, y_ref):
    y_ref[...] = (x1_ref[...]
                  + w_ref[:, 0:1] * g0_ref[...]
                  + w_ref[:, 1:2] * g1_ref[...])


def _sc_scatter(h2, idx):
    @functools.partial(
        pl.kernel,
        mesh=plsc.VectorSubcoreMesh(core_axis_name="c", subcore_axis_name="s"),
        out_type=jax.ShapeDtypeStruct((_P, _D), jnp.float32),
        scratch_types=[
            pltpu.VMEM((128,), jnp.int32),
            pltpu.VMEM((128, _D), jnp.float32),
            pltpu.SemaphoreType.DMA,
        ],
    )
    def k(h2_hbm, idx_hbm, out_hbm, idx_v, rows_v, sem):
        wid = lax.axis_index("s") * 2 + lax.axis_index("c")
        src = (wid % 16) * 128
        pltpu.sync_copy(idx_hbm.at[wid], idx_v)
        pltpu.sync_copy(h2_hbm.at[pl.ds(src, 128)], rows_v)
        pltpu.async_copy(rows_v, out_hbm.at[idx_v], sem).wait()

    return k(h2, idx)


def _sc_gather(f, idx):
    @functools.partial(
        pl.kernel,
        mesh=plsc.VectorSubcoreMesh(core_axis_name="c", subcore_axis_name="s"),
        out_type=jax.ShapeDtypeStruct((_N * _KK, _D), jnp.float32),
        scratch_types=[
            pltpu.VMEM((128,), jnp.int32),
            pltpu.VMEM((128, _D), jnp.float32),
            pltpu.SemaphoreType.DMA,
        ],
    )
    def k(f_hbm, idx_hbm, out_hbm, idx_v, rows_v, sem):
        wid = lax.axis_index("s") * 2 + lax.axis_index("c")
        pltpu.sync_copy(idx_hbm.at[wid], idx_v)
        pltpu.async_copy(f_hbm.at[idx_v], rows_v, sem).wait()
        pltpu.sync_copy(rows_v, out_hbm.at[pl.ds(wid * 128, 128)])

    return k(f, idx)


def kernel(x, norm1_w, norm2_w, Wq, Wk, Wv, Wo, Wg, Wmoe, W1, W2):
    x2d = x.reshape(_N, _D)
    n1 = norm1_w.reshape(1, _D)
    n2 = norm2_w.reshape(1, _D)
    wm_pad = jnp.pad(Wmoe, ((0, 128 - _E), (0, 0)))

    q, k, v, g = pl.pallas_call(
        _pre_body,
        grid=(_N // _TB,),
        in_specs=[
            pl.BlockSpec((_TB, _D), lambda t: (t, 0)),
            pl.BlockSpec((1, _D), lambda t: (0, 0)),
            pl.BlockSpec((_D, _D), lambda t: (0, 0)),
            pl.BlockSpec((_D, _D), lambda t: (0, 0)),
            pl.BlockSpec((_D, _D), lambda t: (0, 0)),
            pl.BlockSpec((_D, _D), lambda t: (0, 0)),
        ],
        out_specs=[pl.BlockSpec((_H, _TB, _HD), lambda t: (0, t, 0))] * 3
        + [pl.BlockSpec((_TB, _D), lambda t: (t, 0))],
        out_shape=[jax.ShapeDtypeStruct((_H, _N, _HD), _BF)] * 3
        + [jax.ShapeDtypeStruct((_N, _D), jnp.float32)],
    )(x2d, n1, Wq, Wk, Wv, Wg)

    attn3 = pl.pallas_call(
        _attn_body,
        grid=(_H, _N // _QB),
        in_specs=[
            pl.BlockSpec((1, _QB, _HD), lambda h, t: (h, t, 0)),
            pl.BlockSpec((1, _N, _HD), lambda h, t: (h, 0, 0)),
            pl.BlockSpec((1, _N, _HD), lambda h, t: (h, 0, 0)),
        ],
        out_specs=pl.BlockSpec((1, _QB, _HD), lambda h, t: (h, t, 0)),
        out_shape=jax.ShapeDtypeStruct((_H, _N, _HD), _BF),
    )(q, k, v)

    x1, h2, pos_out, w_out, meta = pl.pallas_call(
        _post_body,
        out_shape=[
            jax.ShapeDtypeStruct((_N, _D), jnp.float32),
            jax.ShapeDtypeStruct((_N, _D), jnp.float32),
            jax.ShapeDtypeStruct((_N, 128), jnp.int32),
            jax.ShapeDtypeStruct((_N, 128), jnp.float32),
            jax.ShapeDtypeStruct((128, 128), jnp.int32),
        ],
    )(attn3, g, x2d, n2, Wo, wm_pad)

    idx = jnp.concatenate([pos_out[:, 0], pos_out[:, 1]]).reshape(32, 128)
    be = meta[: _NB, 0]

    xs = _sc_scatter(h2, idx)

    f = pl.pallas_call(
        _ffn_body,
        grid_spec=pltpu.PrefetchScalarGridSpec(
            num_scalar_prefetch=1,
            grid=(_NB,),
            in_specs=[
                pl.BlockSpec((_BLK, _D), lambda b, be_r: (b, 0)),
                pl.BlockSpec((1, _FF, _D), lambda b, be_r: (be_r[b], 0, 0)),
                pl.BlockSpec((1, _D, _FF), lambda b, be_r: (be_r[b], 0, 0)),
            ],
            out_specs=pl.BlockSpec((_BLK, _D), lambda b, be_r: (b, 0)),
        ),
        out_shape=jax.ShapeDtypeStruct((_P, _D), jnp.float32),
    )(be, xs, W1, W2)

    gth = _sc_gather(f, idx)

    y = pl.pallas_call(
        _comb_body,
        grid=(_N // _TB,),
        in_specs=[
            pl.BlockSpec((_TB, _D), lambda t: (t, 0)),
            pl.BlockSpec((_TB, _D), lambda t: (t, 0)),
            pl.BlockSpec((_TB, _D), lambda t: (t, 0)),
            pl.BlockSpec((_TB, 128), lambda t: (t, 0)),
        ],
        out_specs=pl.BlockSpec((_TB, _D), lambda t: (t, 0)),
        out_shape=jax.ShapeDtypeStruct((_N, _D), jnp.float32),
    )(x1, gth[: _N], gth[_N:], w_out)

    return y.reshape(_B, _N, _D)


# speedup vs baseline: 1.1935x; 1.0088x over previous
import functools

import jax
import jax.numpy as jnp
from jax import lax
from jax.experimental import pallas as pl
from jax.experimental.pallas import tpu as pltpu
from jax.experimental.pallas import tpu_sc as plsc

_D, _H, _E, _KK, _FF = 768, 12, 8, 2, 2048
_B, _N = 1, 2048
_HD = _D // _H
_TB = 256
_QB = 1024
_BLK = 128
_P = _N * _KK + _E * _BLK
_NB = _P // _BLK
_NEG = -1e30
_BF = jnp.bfloat16


def _dot_t(a, b):
    return lax.dot_general(a.astype(_BF), b.astype(_BF),
                           (((1,), (1,)), ((), ())),
                           preferred_element_type=jnp.float32)


def _rms(x, w):
    return x / jnp.sqrt(jnp.mean(x * x, axis=1, keepdims=True) + 1e-6) * w


def _attn_body(x_ref, n1_ref, wq_ref, wk_ref, wv_ref, o_ref,
               h_s, k_s, v_s):
    t = pl.program_id(1)

    @pl.when((pl.program_id(0) == 0) & (t == 0))
    def _():
        h_s[...] = _rms(x_ref[...], n1_ref[...]).astype(_BF)

    @pl.when(t == 0)
    def _():
        k_s[...] = _dot_t(h_s[...], wk_ref[...]).astype(_BF)
        v_s[...] = _dot_t(h_s[...], wv_ref[...]).astype(_BF)

    q = _dot_t(h_s[pl.ds(t * _QB, _QB), :], wq_ref[...]).astype(_BF)
    s = _dot_t(q, k_s[...]) * 0.125
    m = jnp.max(s, axis=1, keepdims=True)
    p = jnp.exp(s - m)
    l = jnp.sum(p, axis=1, keepdims=True)
    o = lax.dot_general(p.astype(_BF), v_s[...], (((1,), (0,)), ((), ())),
                        preferred_element_type=jnp.float32)
    o_ref[0] = (o / l).astype(_BF)


def _post_body(a_ref, x_ref, n1_ref, wg_ref, n2_ref, wo_ref, wm_ref,
               x1_ref, h2_ref, pos_ref, w_ref, meta_ref):
    a3 = a_ref[...]
    a = a3.transpose(1, 0, 2).reshape(_N, _D)
    out = _dot_t(a, wo_ref[...])
    g = jax.nn.sigmoid(_dot_t(_rms(x_ref[...], n1_ref[...]), wg_ref[...]))
    x1 = x_ref[...] + out * g
    x1_ref[...] = x1
    h2 = _rms(x1, n2_ref[...])
    h2_ref[...] = h2
    lg = _dot_t(h2, wm_ref[...])
    ji = lax.broadcasted_iota(jnp.int32, (_N, 128), 1)
    l1 = jnp.where(ji < _E, lg, _NEG)
    m1 = jnp.max(l1, axis=1, keepdims=True)
    i1 = jnp.min(jnp.where(l1 == m1, ji, 127), axis=1, keepdims=True)
    l2 = jnp.where(ji == i1, _NEG, l1)
    m2 = jnp.max(l2, axis=1, keepdims=True)
    i2 = jnp.min(jnp.where(l2 == m2, ji, 127), axis=1, keepdims=True)
    e2 = jnp.exp(m2 - m1)
    w1 = 1.0 / (1.0 + e2)
    w2 = e2 / (1.0 + e2)
    a0 = (ji == i1).astype(jnp.int32)
    a1 = (ji == i2).astype(jnp.int32)

    def shift_rows(a, sh):
        return jnp.concatenate(
            [jnp.zeros((sh, 128), a.dtype), a[: _N - sh]], axis=0)

    def cumsum_ex_rows(a):
        a = shift_rows(a, 1)
        sh = 1
        while sh < _N:
            a = a + shift_rows(a, sh)
            sh *= 2
        return a

    c0x = cumsum_ex_rows(a0)
    c1x = cumsum_ex_rows(a1)
    c0 = jnp.sum(a0, axis=0, keepdims=True)
    c1 = jnp.sum(a1, axis=0, keepdims=True)
    pc = ((c0 + c1 + (_BLK - 1)) // _BLK) * _BLK

    def shift_lanes(a, sh):
        return jnp.concatenate(
            [jnp.zeros((a.shape[0], sh), a.dtype), a[:, : 128 - sh]], axis=1)

    off = shift_lanes(pc, 1)
    for sh in (1, 2, 4):
        off = off + shift_lanes(off, sh)
    cum_end = off + pc
    pos0 = jnp.sum(a0 * (c0x + off), axis=1, keepdims=True)
    pos1 = jnp.sum(a1 * (c1x + c0 + off), axis=1, keepdims=True)
    pos_ref[...] = jnp.where(ji == 0, pos0, jnp.where(ji == 1, pos1, 0))
    w_ref[...] = jnp.where(ji == 0, w1, jnp.where(ji == 1, w2, 0.0))
    bi = lax.broadcasted_iota(jnp.int32, (128, 128), 0) * _BLK
    ji2 = lax.broadcasted_iota(jnp.int32, (128, 128), 1)
    ge = jnp.where((ji2 < _E) & (bi >= cum_end), 1, 0)
    be = jnp.minimum(jnp.sum(ge, axis=1, keepdims=True), _E - 1)
    meta_ref[...] = jnp.where(ji2 == 0, be, 0)


def _ffn_body(be_ref, x_ref, w1_ref, w2_ref, o_ref):
    h = _dot_t(x_ref[...], w1_ref[0])
    h = h * jax.nn.sigmoid(h)
    o_ref[...] = _dot_t(h, w2_ref[0])


def _comb_body(x1_ref, g0_ref, g1_ref, w_ref, y_ref):
    y_ref[...] = (x1_ref[...]
                  + w_ref[:, 0:1] * g0_ref[...]
                  + w_ref[:, 1:2] * g1_ref[...])


def _sc_scatter(h2, idx):
    @functools.partial(
        pl.kernel,
        mesh=plsc.VectorSubcoreMesh(core_axis_name="c", subcore_axis_name="s"),
        out_type=jax.ShapeDtypeStruct((_P, _D), jnp.float32),
        scratch_types=[
            pltpu.VMEM((128,), jnp.int32),
            pltpu.VMEM((128, _D), jnp.float32),
            pltpu.SemaphoreType.DMA,
        ],
    )
    def k(h2_hbm, idx_hbm, out_hbm, idx_v, rows_v, sem):
        wid = lax.axis_index("s") * 2 + lax.axis_index("c")
        src = (wid % 16) * 128
        pltpu.sync_copy(idx_hbm.at[wid], idx_v)
        pltpu.sync_copy(h2_hbm.at[pl.ds(src, 128)], rows_v)
        pltpu.async_copy(rows_v, out_hbm.at[idx_v], sem).wait()

    return k(h2, idx)


def _sc_gather(f, idx):
    @functools.partial(
        pl.kernel,
        mesh=plsc.VectorSubcoreMesh(core_axis_name="c", subcore_axis_name="s"),
        out_type=jax.ShapeDtypeStruct((_N * _KK, _D), jnp.float32),
        scratch_types=[
            pltpu.VMEM((128,), jnp.int32),
            pltpu.VMEM((128, _D), jnp.float32),
            pltpu.SemaphoreType.DMA,
        ],
    )
    def k(f_hbm, idx_hbm, out_hbm, idx_v, rows_v, sem):
        wid = lax.axis_index("s") * 2 + lax.axis_index("c")
        pltpu.sync_copy(idx_hbm.at[wid], idx_v)
        pltpu.async_copy(f_hbm.at[idx_v], rows_v, sem).wait()
        pltpu.sync_copy(rows_v, out_hbm.at[pl.ds(wid * 128, 128)])

    return k(f, idx)


def kernel(x, norm1_w, norm2_w, Wq, Wk, Wv, Wo, Wg, Wmoe, W1, W2):
    x2d = x.reshape(_N, _D)
    n1 = norm1_w.reshape(1, _D)
    n2 = norm2_w.reshape(1, _D)
    wm_pad = jnp.pad(Wmoe, ((0, 128 - _E), (0, 0)))

    attn3 = pl.pallas_call(
        _attn_body,
        grid=(_H, _N // _QB),
        in_specs=[
            pl.BlockSpec((_N, _D), lambda h, t: (0, 0)),
            pl.BlockSpec((1, _D), lambda h, t: (0, 0)),
            pl.BlockSpec((_HD, _D), lambda h, t: (h, 0)),
            pl.BlockSpec((_HD, _D), lambda h, t: (h, 0)),
            pl.BlockSpec((_HD, _D), lambda h, t: (h, 0)),
        ],
        out_specs=pl.BlockSpec((1, _QB, _HD), lambda h, t: (h, t, 0)),
        out_shape=jax.ShapeDtypeStruct((_H, _N, _HD), _BF),
        scratch_shapes=[
            pltpu.VMEM((_N, _D), _BF),
            pltpu.VMEM((_N, _HD), _BF),
            pltpu.VMEM((_N, _HD), _BF),
        ],
    )(x2d, n1, Wq, Wk, Wv)

    x1, h2, pos_out, w_out, meta = pl.pallas_call(
        _post_body,
        out_shape=[
            jax.ShapeDtypeStruct((_N, _D), jnp.float32),
            jax.ShapeDtypeStruct((_N, _D), jnp.float32),
            jax.ShapeDtypeStruct((_N, 128), jnp.int32),
            jax.ShapeDtypeStruct((_N, 128), jnp.float32),
            jax.ShapeDtypeStruct((128, 128), jnp.int32),
        ],
    )(attn3, x2d, n1, Wg, n2, Wo, wm_pad)

    idx = jnp.concatenate([pos_out[:, 0], pos_out[:, 1]]).reshape(32, 128)
    be = meta[: _NB, 0]

    xs = _sc_scatter(h2, idx)

    f = pl.pallas_call(
        _ffn_body,
        grid_spec=pltpu.PrefetchScalarGridSpec(
            num_scalar_prefetch=1,
            grid=(_NB,),
            in_specs=[
                pl.BlockSpec((_BLK, _D), lambda b, be_r: (b, 0)),
                pl.BlockSpec((1, _FF, _D), lambda b, be_r: (be_r[b], 0, 0)),
                pl.BlockSpec((1, _D, _FF), lambda b, be_r: (be_r[b], 0, 0)),
            ],
            out_specs=pl.BlockSpec((_BLK, _D), lambda b, be_r: (b, 0)),
        ),
        out_shape=jax.ShapeDtypeStruct((_P, _D), jnp.float32),
    )(be, xs, W1, W2)

    gth = _sc_gather(f, idx)

    y = pl.pallas_call(
        _comb_body,
        grid=(_N // _TB,),
        in_specs=[
            pl.BlockSpec((_TB, _D), lambda t: (t, 0)),
            pl.BlockSpec((_TB, _D), lambda t: (t, 0)),
            pl.BlockSpec((_TB, _D), lambda t: (t + _N // _TB, 0)),
            pl.BlockSpec((_TB, 128), lambda t: (t, 0)),
        ],
        out_specs=pl.BlockSpec((_TB, _D), lambda t: (t, 0)),
        out_shape=jax.ShapeDtypeStruct((_N, _D), jnp.float32),
    )(x1, gth, gth, w_out)

    return y.reshape(_B, _N, _D)


# speedup vs baseline: 1.3457x; 1.1275x over previous
import functools

import jax
import jax.numpy as jnp
from jax import lax
from jax.experimental import pallas as pl
from jax.experimental.pallas import tpu as pltpu
from jax.experimental.pallas import tpu_sc as plsc

_D, _H, _E, _KK, _FF = 768, 12, 8, 2, 2048
_B, _N = 1, 2048
_HD = _D // _H
_TB = 256
_QB = 1024
_BLK = 256
_P = _N * _KK + _E * _BLK
_NB = _P // _BLK
_NEG = -1e30
_BF = jnp.bfloat16


def _dot_t(a, b):
    return lax.dot_general(a.astype(_BF), b.astype(_BF),
                           (((1,), (1,)), ((), ())),
                           preferred_element_type=jnp.float32)


def _rms(x, w):
    return x / jnp.sqrt(jnp.mean(x * x, axis=1, keepdims=True) + 1e-6) * w


def _attn_body(x_ref, n1_ref, wq_ref, wk_ref, wv_ref, o_ref,
               h_s, k_s, v_s):
    t = pl.program_id(1)

    @pl.when((pl.program_id(0) == 0) & (t == 0))
    def _():
        h_s[...] = _rms(x_ref[...], n1_ref[...]).astype(_BF)

    @pl.when(t == 0)
    def _():
        k_s[...] = _dot_t(h_s[...], wk_ref[...]).astype(_BF)
        v_s[...] = _dot_t(h_s[...], wv_ref[...]).astype(_BF)

    q = _dot_t(h_s[pl.ds(t * _QB, _QB), :], wq_ref[...]).astype(_BF)
    s = _dot_t(q, k_s[...]) * 0.125
    m = jnp.max(s, axis=1, keepdims=True)
    p = jnp.exp(s - m)
    l = jnp.sum(p, axis=1, keepdims=True)
    o = lax.dot_general(p.astype(_BF), v_s[...], (((1,), (0,)), ((), ())),
                        preferred_element_type=jnp.float32)
    o_ref[0] = (o / l).astype(_BF)


def _post_body(a_ref, x_ref, n1_ref, wg_ref, n2_ref, wo_ref, wm_ref,
               x1_ref, h2_ref, pos_ref, w_ref, meta_ref):
    a3 = a_ref[...]
    a = a3.transpose(1, 0, 2).reshape(_N, _D)
    out = _dot_t(a, wo_ref[...])
    g = jax.nn.sigmoid(_dot_t(_rms(x_ref[...], n1_ref[...]), wg_ref[...]))
    x1 = x_ref[...] + out * g
    x1_ref[...] = x1
    h2 = _rms(x1, n2_ref[...])
    h2_ref[...] = h2
    lg = _dot_t(h2, wm_ref[...])
    ji = lax.broadcasted_iota(jnp.int32, (_N, 128), 1)
    l1 = jnp.where(ji < _E, lg, _NEG)
    m1 = jnp.max(l1, axis=1, keepdims=True)
    i1 = jnp.min(jnp.where(l1 == m1, ji, 127), axis=1, keepdims=True)
    l2 = jnp.where(ji == i1, _NEG, l1)
    m2 = jnp.max(l2, axis=1, keepdims=True)
    i2 = jnp.min(jnp.where(l2 == m2, ji, 127), axis=1, keepdims=True)
    e2 = jnp.exp(m2 - m1)
    w1 = 1.0 / (1.0 + e2)
    w2 = e2 / (1.0 + e2)
    a0 = (ji == i1).astype(jnp.int32)
    a1 = (ji == i2).astype(jnp.int32)

    def shift_rows(a, sh):
        return jnp.concatenate(
            [jnp.zeros((sh, 128), a.dtype), a[: _N - sh]], axis=0)

    def cumsum_ex_rows(a):
        a = shift_rows(a, 1)
        sh = 1
        while sh < _N:
            a = a + shift_rows(a, sh)
            sh *= 2
        return a

    c0x = cumsum_ex_rows(a0)
    c1x = cumsum_ex_rows(a1)
    c0 = jnp.sum(a0, axis=0, keepdims=True)
    c1 = jnp.sum(a1, axis=0, keepdims=True)
    pc = ((c0 + c1 + (_BLK - 1)) // _BLK) * _BLK

    def shift_lanes(a, sh):
        return jnp.concatenate(
            [jnp.zeros((a.shape[0], sh), a.dtype), a[:, : 128 - sh]], axis=1)

    off = shift_lanes(pc, 1)
    for sh in (1, 2, 4):
        off = off + shift_lanes(off, sh)
    cum_end = off + pc
    pos0 = jnp.sum(a0 * (c0x + off), axis=1, keepdims=True)
    pos1 = jnp.sum(a1 * (c1x + c0 + off), axis=1, keepdims=True)
    pos_ref[...] = jnp.where(ji == 0, pos0, jnp.where(ji == 1, pos1, 0))
    w_ref[...] = jnp.where(ji == 0, w1, jnp.where(ji == 1, w2, 0.0))
    bi = lax.broadcasted_iota(jnp.int32, (128, 128), 0) * _BLK
    ji2 = lax.broadcasted_iota(jnp.int32, (128, 128), 1)
    ge = jnp.where((ji2 < _E) & (bi >= cum_end), 1, 0)
    be = jnp.minimum(jnp.sum(ge, axis=1, keepdims=True), _E - 1)
    meta_ref[...] = jnp.where(ji2 == 0, be, 0)


def _ffn_body(be_ref, x_ref, w1_ref, w2_ref, o_ref):
    h = _dot_t(x_ref[...], w1_ref[0])
    h = h * jax.nn.sigmoid(h)
    o_ref[...] = _dot_t(h, w2_ref[0])


def _comb_body(x1_ref, g0_ref, g1_ref, w_ref, y_ref):
    y_ref[...] = (x1_ref[...]
                  + w_ref[:, 0:1] * g0_ref[...]
                  + w_ref[:, 1:2] * g1_ref[...])


def _sc_scatter(h2, idx):
    @functools.partial(
        pl.kernel,
        mesh=plsc.VectorSubcoreMesh(core_axis_name="c", subcore_axis_name="s"),
        out_type=jax.ShapeDtypeStruct((_P, _D), jnp.float32),
        scratch_types=[
            pltpu.VMEM((128,), jnp.int32),
            pltpu.VMEM((128, _D), jnp.float32),
            pltpu.SemaphoreType.DMA,
        ],
    )
    def k(h2_hbm, idx_hbm, out_hbm, idx_v, rows_v, sem):
        wid = lax.axis_index("s") * 2 + lax.axis_index("c")
        src = (wid % 16) * 128
        pltpu.sync_copy(idx_hbm.at[wid], idx_v)
        pltpu.sync_copy(h2_hbm.at[pl.ds(src, 128)], rows_v)
        pltpu.async_copy(rows_v, out_hbm.at[idx_v], sem).wait()

    return k(h2, idx)


def _sc_gather(f, idx):
    @functools.partial(
        pl.kernel,
        mesh=plsc.VectorSubcoreMesh(core_axis_name="c", subcore_axis_name="s"),
        out_type=jax.ShapeDtypeStruct((_N * _KK, _D), jnp.float32),
        scratch_types=[
            pltpu.VMEM((128,), jnp.int32),
            pltpu.VMEM((128, _D), jnp.float32),
            pltpu.SemaphoreType.DMA,
        ],
    )
    def k(f_hbm, idx_hbm, out_hbm, idx_v, rows_v, sem):
        wid = lax.axis_index("s") * 2 + lax.axis_index("c")
        pltpu.sync_copy(idx_hbm.at[wid], idx_v)
        pltpu.async_copy(f_hbm.at[idx_v], rows_v, sem).wait()
        pltpu.sync_copy(rows_v, out_hbm.at[pl.ds(wid * 128, 128)])

    return k(f, idx)


def kernel(x, norm1_w, norm2_w, Wq, Wk, Wv, Wo, Wg, Wmoe, W1, W2):
    x2d = x.reshape(_N, _D)
    n1 = norm1_w.reshape(1, _D)
    n2 = norm2_w.reshape(1, _D)
    wm_pad = jnp.pad(Wmoe, ((0, 128 - _E), (0, 0)))

    attn3 = pl.pallas_call(
        _attn_body,
        grid=(_H, _N // _QB),
        in_specs=[
            pl.BlockSpec((_N, _D), lambda h, t: (0, 0)),
            pl.BlockSpec((1, _D), lambda h, t: (0, 0)),
            pl.BlockSpec((_HD, _D), lambda h, t: (h, 0)),
            pl.BlockSpec((_HD, _D), lambda h, t: (h, 0)),
            pl.BlockSpec((_HD, _D), lambda h, t: (h, 0)),
        ],
        out_specs=pl.BlockSpec((1, _QB, _HD), lambda h, t: (h, t, 0)),
        out_shape=jax.ShapeDtypeStruct((_H, _N, _HD), _BF),
        scratch_shapes=[
            pltpu.VMEM((_N, _D), _BF),
            pltpu.VMEM((_N, _HD), _BF),
            pltpu.VMEM((_N, _HD), _BF),
        ],
    )(x2d, n1, Wq, Wk, Wv)

    x1, h2, pos_out, w_out, meta = pl.pallas_call(
        _post_body,
        out_shape=[
            jax.ShapeDtypeStruct((_N, _D), jnp.float32),
            jax.ShapeDtypeStruct((_N, _D), jnp.float32),
            jax.ShapeDtypeStruct((_N, 128), jnp.int32),
            jax.ShapeDtypeStruct((_N, 128), jnp.float32),
            jax.ShapeDtypeStruct((128, 128), jnp.int32),
        ],
    )(attn3, x2d, n1, Wg, n2, Wo, wm_pad)

    idx = jnp.concatenate([pos_out[:, 0], pos_out[:, 1]]).reshape(32, 128)
    be = meta[: _NB, 0]

    xs = _sc_scatter(h2, idx)

    f = pl.pallas_call(
        _ffn_body,
        grid_spec=pltpu.PrefetchScalarGridSpec(
            num_scalar_prefetch=1,
            grid=(_NB,),
            in_specs=[
                pl.BlockSpec((_BLK, _D), lambda b, be_r: (b, 0)),
                pl.BlockSpec((1, _FF, _D), lambda b, be_r: (be_r[b], 0, 0)),
                pl.BlockSpec((1, _D, _FF), lambda b, be_r: (be_r[b], 0, 0)),
            ],
            out_specs=pl.BlockSpec((_BLK, _D), lambda b, be_r: (b, 0)),
        ),
        out_shape=jax.ShapeDtypeStruct((_P, _D), jnp.float32),
    )(be, xs, W1, W2)

    gth = _sc_gather(f, idx)

    y = pl.pallas_call(
        _comb_body,
        grid=(_N // _TB,),
        in_specs=[
            pl.BlockSpec((_TB, _D), lambda t: (t, 0)),
            pl.BlockSpec((_TB, _D), lambda t: (t, 0)),
            pl.BlockSpec((_TB, _D), lambda t: (t + _N // _TB, 0)),
            pl.BlockSpec((_TB, 128), lambda t: (t, 0)),
        ],
        out_specs=pl.BlockSpec((_TB, _D), lambda t: (t, 0)),
        out_shape=jax.ShapeDtypeStruct((_N, _D), jnp.float32),
    )(x1, gth, gth, w_out)

    return y.reshape(_B, _N, _D)
